# Initial kernel scaffold; baseline (speedup 1.0000x reference)
import jax
import jax.numpy as jnp
from jax.experimental import pallas as pl


def kernel(x, nodes, feats, neighs0, neighs1, neighs1_0, W0, b0, W1, b1):
    raise NotImplementedError("write your pallas kernel here")


# speedup vs baseline: 4.1895x; 4.1895x over previous
import jax
import jax.numpy as jnp
from jax import lax
from jax.experimental import pallas as pl
from jax.experimental.pallas import tpu as pltpu
from jax.experimental.pallas import tpu_sc as plsc

B = 1024
N = 50000
D = 256
N0 = 16
N1 = 16
L1 = 256
L2 = 128
R = B * N1
NC, NS = 2, 16
NW = NC * NS
RW = R // NW
BW = B // NW
G = 8
CH = G * N0

_PREC = jax.lax.Precision.HIGHEST
_LANES = D // 16


def _emit_segment_sum(feats, idx_v, out_hbm, out_row0, ndst, bufs, sems,
                      obuf, osem):
    nch = (ndst * N0) // CH
    niter = nch // 2

    def gather(ch, buf, sem):
        return pltpu.async_copy(
            feats.at[idx_v.at[pl.ds(ch * CH, CH)]], buf, sem)

    def reduce_buf(buf, obase):
        def red(m, _):
            row = m * N0
            acc = [buf[row, pl.ds(c * 16, 16)] for c in range(_LANES)]
            for j in range(1, N0):
                for c in range(_LANES):
                    acc[c] += buf[row + j, pl.ds(c * 16, 16)]
            for c in range(_LANES):
                obuf[obase + m, pl.ds(c * 16, 16)] = acc[c]
            return 0

        lax.fori_loop(0, G, red, 0, unroll=False)

    gather(0, bufs[0], sems[0])
    gather(1, bufs[1], sems[1])

    def body(i, _):
        @pl.when(i > 0)
        def _():
            pltpu.make_async_copy(
                out_hbm.at[pl.ds(0, 2 * G), :], obuf, osem).wait()

        pltpu.make_async_copy(feats.at[pl.ds(0, CH), :], bufs[0],
                              sems[0]).wait()
        reduce_buf(bufs[0], 0)

        @pl.when(i < niter - 1)
        def _():
            gather(2 * i + 2, bufs[0], sems[0])

        pltpu.make_async_copy(feats.at[pl.ds(0, CH), :], bufs[1],
                              sems[1]).wait()
        reduce_buf(bufs[1], G)

        @pl.when(i < niter - 1)
        def _():
            gather(2 * i + 3, bufs[1], sems[1])

        pltpu.async_copy(obuf, out_hbm.at[pl.ds(out_row0 + i * 2 * G, 2 * G), :],
                         osem)
        return 0

    lax.fori_loop(0, niter, body, 0, unroll=False)
    pltpu.make_async_copy(out_hbm.at[pl.ds(0, 2 * G), :], obuf, osem).wait()


def _sc_gather_kernel(feats, idx0w, n1w, idx10w, sum0_out, hp2_out,
                      sum10_out, idxa, idxb, idxc, buf0, buf1, obuf,
                      sem0, sem1, osem):
    wid = lax.axis_index("s") * NC + lax.axis_index("c")
    bufs = (buf0, buf1)
    sems = (sem0, sem1)

    pltpu.sync_copy(idx10w.at[wid], idxa)
    pltpu.sync_copy(n1w.at[wid], idxb)
    pltpu.sync_copy(idx0w.at[wid], idxc)

    _emit_segment_sum(feats, idxa, sum10_out, wid * RW, RW, bufs, sems,
                      obuf, osem)

    nch = RW // CH
    cps = [pltpu.async_copy(feats.at[idxb.at[pl.ds(k * CH, CH)]],
                            bufs[k % 2], sems[k % 2])
           for k in range(2)]
    for k in range(nch):
        cps[k % 2].wait()
        pltpu.sync_copy(bufs[k % 2],
                        hp2_out.at[pl.ds(wid * RW + k * CH, CH), :])
        if k + 2 < nch:
            cps[k % 2] = pltpu.async_copy(
                feats.at[idxb.at[pl.ds((k + 2) * CH, CH)]],
                bufs[k % 2], sems[k % 2])

    _emit_segment_sum(feats, idxc, sum0_out, wid * BW, BW, bufs, sems,
                      obuf, osem)


@jax.jit
def _sc_gather(feats, idx0w, n1w, idx10w):
    mesh = plsc.VectorSubcoreMesh(core_axis_name="c", subcore_axis_name="s",
                                  num_cores=NC, num_subcores=NS)
    return pl.kernel(
        _sc_gather_kernel,
        out_type=[
            jax.ShapeDtypeStruct((B, D), jnp.float32),
            jax.ShapeDtypeStruct((R, D), jnp.float32),
            jax.ShapeDtypeStruct((R, D), jnp.float32),
        ],
        mesh=mesh,
        scratch_types=[
            pltpu.VMEM((RW * N0,), jnp.int32),
            pltpu.VMEM((RW,), jnp.int32),
            pltpu.VMEM((BW * N0,), jnp.int32),
            pltpu.VMEM((CH, D), jnp.float32),
            pltpu.VMEM((CH, D), jnp.float32),
            pltpu.VMEM((2 * G, D), jnp.float32),
            pltpu.SemaphoreType.DMA,
            pltpu.SemaphoreType.DMA,
            pltpu.SemaphoreType.DMA,
        ],
    )(feats, idx0w, n1w, idx10w)


def _h2_body(hp2_ref, s10_ref, w0_ref, b0_ref, out_ref):
    h = jnp.dot(hp2_ref[...], w0_ref[:D], preferred_element_type=jnp.float32,
                precision=_PREC)
    h += jnp.dot(s10_ref[...] * (1.0 / N0), w0_ref[D:],
                 preferred_element_type=jnp.float32, precision=_PREC)
    h += b0_ref[...][None, :]
    h = jnp.maximum(h, 0.0)
    blk = h.shape[0] // N1
    out_ref[...] = jnp.sum(h.reshape(blk, N1, L1), axis=1)


def _final_body(x_ref, s0_ref, ssum_ref, w0_ref, b0_ref, w1_ref, b1_ref,
                out_ref):
    hp = jnp.dot(x_ref[...], w0_ref[:D], preferred_element_type=jnp.float32,
                 precision=_PREC)
    hp += jnp.dot(s0_ref[...] * (1.0 / N0), w0_ref[D:],
                  preferred_element_type=jnp.float32, precision=_PREC)
    hp += b0_ref[...][None, :]
    hp = jnp.maximum(hp, 0.0)
    o = jnp.dot(hp, w1_ref[:L1], preferred_element_type=jnp.float32,
                precision=_PREC)
    o += jnp.dot(ssum_ref[...] * (1.0 / N1), w1_ref[L1:],
                 preferred_element_type=jnp.float32, precision=_PREC)
    o += b1_ref[...][None, :]
    out_ref[...] = jnp.maximum(o, 0.0)


_H2_BLK = 2048


@jax.jit
def _tc_forward(x, sum0, hp2, sum10, W0, b0, W1, b1):
    nblk = R // _H2_BLK
    ssum = pl.pallas_call(
        _h2_body,
        grid=(nblk,),
        in_specs=[
            pl.BlockSpec((_H2_BLK, D), lambda i: (i, 0)),
            pl.BlockSpec((_H2_BLK, D), lambda i: (i, 0)),
            pl.BlockSpec((2 * D, L1), lambda i: (0, 0)),
            pl.BlockSpec((L1,), lambda i: (0,)),
        ],
        out_specs=pl.BlockSpec((_H2_BLK // N1, L1), lambda i: (i, 0)),
        out_shape=jax.ShapeDtypeStruct((B, L1), jnp.float32),
    )(hp2, sum10, W0, b0)
    return pl.pallas_call(
        _final_body,
        out_shape=jax.ShapeDtypeStruct((B, L2), jnp.float32),
    )(x, sum0, ssum, W0, b0, W1, b1)


def kernel(x, nodes, feats, neighs0, neighs1, neighs1_0, W0, b0, W1, b1):
    idx0w = neighs0.reshape(NW, BW * N0)
    n1w = neighs1.reshape(NW, RW)
    idx10w = neighs1_0.reshape(NW, RW * N0)
    sum0, hp2, sum10 = _sc_gather(feats, idx0w, n1w, idx10w)
    return _tc_forward(x, sum0, hp2, sum10, W0, b0, W1, b1)
